# baseline (speedup 1.0000x reference)
import functools

import jax
import jax.numpy as jnp
from jax import lax
from jax.experimental import pallas as pl
from jax.experimental.pallas import tpu as pltpu
from jax.experimental.pallas import tpu_sc as plsc

_DIM = 64
_CTX = 4
_K = 6
_NC = 2
_NS = 16
_NW = _NC * _NS
_L = 16


def _pad_body(x_ref, out_ref):
    out_ref[:, 0:_DIM] = x_ref[...]


def _pad_rows(x, rows_out, rb):
    return pl.pallas_call(
        _pad_body,
        grid=(rows_out // rb,),
        in_specs=[pl.BlockSpec((rb, _DIM), lambda i: (i, 0))],
        out_specs=pl.BlockSpec((rb, 128), lambda i: (i, 0)),
        out_shape=jax.ShapeDtypeStruct((rows_out, 128), jnp.float32),
    )(x)


def _transpose_body(wp_ref, out_ref):
    out_ref[:, 0:_DIM] = wp_ref[...].T


def _transpose_wp(WP, rows_out, cb):
    return pl.pallas_call(
        _transpose_body,
        grid=(rows_out // cb,),
        in_specs=[pl.BlockSpec((_DIM, cb), lambda i: (0, i))],
        out_specs=pl.BlockSpec((cb, 128), lambda i: (i, 0)),
        out_shape=jax.ShapeDtypeStruct((rows_out, 128), jnp.float32),
    )(WP)


def _make_sc_call(B):
    bpw = B // _NW
    n_ctx_chunks = bpw * _CTX // 128
    n_tgt_chunks = bpw * _K // 128
    bpc = 128 // _CTX
    gpc = 128 // _L
    mesh = plsc.VectorSubcoreMesh(core_axis_name="c", subcore_axis_name="s")

    @functools.partial(
        pl.kernel,
        mesh=mesh,
        compiler_params=pltpu.CompilerParams(
            needs_layout_passes=False, use_tc_tiling_on_sc=True),
        out_type=jax.ShapeDtypeStruct((B * _K,), jnp.float32),
        scratch_types=[
            pltpu.VMEM((128,), jnp.int32),
            pltpu.VMEM((n_ctx_chunks * 128,), jnp.int32),
            pltpu.VMEM((n_tgt_chunks * 128,), jnp.int32),
            pltpu.VMEM((bpw, 128), jnp.float32),
            pltpu.VMEM((128, 128), jnp.float32),
            pltpu.VMEM((128, 128), jnp.float32),
            pltpu.VMEM((128, 128), jnp.float32),
            pltpu.VMEM((128, 128), jnp.float32),
            pltpu.VMEM((bpw * _K,), jnp.float32),
            pltpu.SemaphoreType.DMA,
            pltpu.SemaphoreType.DMA,
            pltpu.SemaphoreType.DMA,
            pltpu.SemaphoreType.DMA,
            pltpu.SemaphoreType.DMA,
        ],
    )
    def sc_kernel(ctx_hbm, doc_hbm, tgt_hbm, d_hbm, w_hbm, wpt_hbm, out_hbm,
                  doc_idx, ctx_idx, tgt_idx, d_rows, wb0, wb1, pb0, pb1,
                  out_v, sem_d, sem_w0, sem_w1, sem_p0, sem_p1):
        wid = lax.axis_index("s") * _NC + lax.axis_index("c")
        base = wid * bpw
        wbufs = (wb0, wb1)
        wsems = (sem_w0, sem_w1)
        pbufs = (pb0, pb1)
        psems = (sem_p0, sem_p1)

        pltpu.sync_copy(doc_hbm.at[pl.ds(base, bpw)], doc_idx)
        pltpu.sync_copy(
            ctx_hbm.at[pl.ds(wid * n_ctx_chunks * 128, n_ctx_chunks * 128)],
            ctx_idx)
        pltpu.sync_copy(
            tgt_hbm.at[pl.ds(wid * n_tgt_chunks * 128, n_tgt_chunks * 128)],
            tgt_idx)

        def start_w(c):
            return pltpu.async_copy(
                w_hbm.at[ctx_idx.at[pl.ds(c * 128, 128)]],
                wbufs[c % 2], wsems[c % 2])

        def start_p(t):
            return pltpu.async_copy(
                wpt_hbm.at[tgt_idx.at[pl.ds(t * 128, 128)]],
                pbufs[t % 2], psems[t % 2])

        cp_d = pltpu.async_copy(d_hbm.at[doc_idx], d_rows, sem_d)
        cp_w = [start_w(0), start_w(1)]
        cp_p = [start_p(0), start_p(1)]
        cp_d.wait()

        for c in range(n_ctx_chunks):
            cp_w[c].wait()
            wb = wbufs[c % 2]

            def hbody(bl, carry, c=c, wb=wb):
                b = c * bpc + bl
                for j in range(_DIM // _L):
                    h = d_rows[b, pl.ds(j * _L, _L)]
                    for cc in range(_CTX):
                        h = h + wb[bl * _CTX + cc, pl.ds(j * _L, _L)]
                    d_rows[b, pl.ds(j * _L, _L)] = h
                return carry

            lax.fori_loop(0, bpc, hbody, 0)
            if c + 2 < n_ctx_chunks:
                cp_w.append(start_w(c + 2))

        lanes = lax.iota(jnp.int32, _L)
        for t in range(n_tgt_chunks):
            cp_p[t].wait()
            pb = pbufs[t % 2]

            def obody(gl, carry, t=t, pb=pb):
                rows = jnp.full((_L,), t * 128 + gl * _L, jnp.int32) + lanes
                lrows = rows - t * 128
                bs = lax.div(rows, jnp.full((_L,), _K, jnp.int32))
                acc = jnp.zeros((_L,), jnp.float32)
                for d in range(_DIM):
                    dd = jnp.full((_L,), d, jnp.int32)
                    hv = plsc.load_gather(d_rows, [bs, dd])
                    wv = plsc.load_gather(pb, [lrows, dd])
                    acc = acc + hv * wv
                out_v[pl.ds(t * 128 + gl * _L, _L)] = acc
                return carry

            lax.fori_loop(0, gpc, obody, 0)
            if t + 2 < n_tgt_chunks:
                cp_p.append(start_p(t + 2))

        pltpu.sync_copy(out_v, out_hbm.at[pl.ds(base * _K, bpw * _K)])

    return sc_kernel


def kernel(ctxs, docs, targets, D, W, WP):
    B = ctxs.shape[0]
    n_docs = D.shape[0]
    n_w = W.shape[0]
    n_words = WP.shape[1]
    cb = 2048
    rows_out = ((n_words + cb - 1) // cb) * cb
    rb = 2000

    ctx_flat = ctxs.reshape(-1)
    tgt_flat = targets.reshape(-1)
    D2 = _pad_rows(D, ((n_docs + rb - 1) // rb) * rb, rb)
    W2 = _pad_rows(W, ((n_w + rb - 1) // rb) * rb, rb)
    WPT2 = _transpose_wp(WP, rows_out, cb)
    out_flat = _make_sc_call(B)(ctx_flat, docs, tgt_flat, D2, W2, WPT2)
    return out_flat.reshape(B, _K)
